# baseline (speedup 1.0000x reference)
import functools

import jax
import jax.numpy as jnp
from jax.experimental import pallas as pl

N = 5000
D_FEAT = 256
KP1 = 33
E = N * KP1
TILE = 128
NT = (N + TILE - 1) // TILE
NP = NT * TILE
KW = 64
ETILE = 1024
EG = (E + ETILE - 1) // ETILE
EPS_BN = 1e-5
HIGHEST = jax.lax.Precision.HIGHEST


def _knn_kernel(xt_ref, xft_ref, lt_ref, ltT_ref, lf_ref, idx_ref, feat_ref):
    xt = xt_ref[...]
    xft = xft_ref[...]
    lt = lt_ref[...]
    ltT = ltT_ref[...]
    lf = lf_ref[...]

    px_r = ltT[0:1, :]
    py_r = ltT[1:2, :]

    sq_r = px_r * px_r + py_r * py_r
    pxt = lt[:, 0:1]
    pyt = lt[:, 1:2]
    sq_t = pxt * pxt + pyt * pyt

    pxt_b = pxt.astype(jnp.bfloat16).astype(jnp.float32)
    pyt_b = pyt.astype(jnp.bfloat16).astype(jnp.float32)
    pxr_b = px_r.astype(jnp.bfloat16).astype(jnp.float32)
    pyr_b = py_r.astype(jnp.bfloat16).astype(jnp.float32)
    g = pxt_b * pxr_b + pyt_b * pyr_b
    d = sq_t + sq_r - 2.0 * g
    neg = -d

    S_T = jax.lax.dot_general(xft, xt, (((0,), (1,)), ((), ())),
                              precision=HIGHEST)

    na = jnp.maximum(
        jnp.sqrt(jnp.sum(xt * xt, axis=1, keepdims=True)), 1e-8)
    xft_sq = xft * xft
    ns_col = jax.lax.dot_general(xft_sq, jnp.ones((D_FEAT, 1), jnp.float32),
                                 (((0,), (0,)), ((), ())))
    nb_col = jnp.maximum(jnp.sqrt(ns_col), 1e-8)

    tab5 = jnp.concatenate([lf, nb_col], axis=1)
    tab_hi = tab5.astype(jnp.bfloat16)
    r1_ = tab5 - tab_hi.astype(jnp.float32)
    tab_mid = r1_.astype(jnp.bfloat16)
    tab_lo = (r1_ - tab_mid.astype(jnp.float32)).astype(jnp.bfloat16)
    st_hi = S_T.astype(jnp.bfloat16)
    st_lo = (S_T - st_hi.astype(jnp.float32)).astype(jnp.bfloat16)
    tab = jnp.concatenate([tab_hi, tab_mid, tab_lo, st_hi, st_lo],
                          axis=1)

    cols = jax.lax.broadcasted_iota(jnp.int32, (TILE, N), 1)
    lane = jax.lax.broadcasted_iota(jnp.int32, (TILE, KW), 1)
    dmask = (jax.lax.broadcasted_iota(jnp.int32, (TILE, TILE), 0) ==
             jax.lax.broadcasted_iota(jnp.int32, (TILE, TILE), 1))

    tx, ty = pxt, pyt
    tw = lt[:, 2:3]
    th = lt[:, 3:4]

    BIG = jnp.int32(1 << 30)
    NEG = jnp.float32(-3.0e38)

    def body(kk, carry):
        neg, idx_acc, f_acc = carry
        m = jnp.max(neg, axis=1, keepdims=True)
        tiebreak = jnp.where(neg == m, cols, BIG)
        ji = jnp.min(tiebreak, axis=1, keepdims=True)
        mask = cols == ji
        neg = jnp.where(mask, NEG, neg)

        oh = mask.astype(jnp.bfloat16)
        res = jax.lax.dot_general(oh, tab, (((1,), (0,)), ((), ())),
                                  preferred_element_type=jnp.float32)

        def _rec(c):
            return (res[:, c:c + 1] + res[:, c + 5:c + 6]) + res[:, c + 10:c + 11]

        sx = _rec(0)
        sy = _rec(1)
        sw = _rec(2)
        sh = _rec(3)
        nb = _rec(4)
        dot = (jnp.sum(jnp.where(dmask, res[:, 15:15 + TILE], 0.0),
                       axis=1, keepdims=True) +
               jnp.sum(jnp.where(dmask, res[:, 15 + TILE:15 + 2 * TILE], 0.0),
                       axis=1, keepdims=True))

        hsum = sh + th + 1e-8
        f1 = 2.0 * (sx - tx) / hsum
        f2 = 2.0 * (sy - ty) / hsum
        f3 = jnp.log(sh / (th + 1e-8))
        f4 = jnp.log(sw / (tw + 1e-8))

        sx1 = sx - sw / 2
        sy1 = sy - sh / 2
        sx2 = sx + sw / 2
        sy2 = sy + sh / 2
        tx1 = tx - tw / 2
        ty1 = ty - th / 2
        tx2 = tx + tw / 2
        ty2 = ty + th / 2
        ix1 = jnp.maximum(sx1, tx1)
        iy1 = jnp.maximum(sy1, ty1)
        ix2 = jnp.minimum(sx2, tx2)
        iy2 = jnp.minimum(sy2, ty2)
        inter = jnp.clip(ix2 - ix1, 0.0, None) * jnp.clip(iy2 - iy1, 0.0, None)
        sa = sw * sh
        ta = tw * th
        union = sa + ta - inter
        ixc = (ix1 + ix2) / 2
        iyc = (iy1 + iy2) / 2
        uxc = (sx * sa + tx * ta) / (union + 1e-8)
        uyc = (sy * sa + ty * ta) / (union + 1e-8)
        dist = jnp.sqrt((ixc - uxc) ** 2 + (iyc - uyc) ** 2)
        f5 = inter / (union + 1e-8) - dist ** 2 / (union + 1e-8)

        f6 = dot / (na * nb)

        ksel = lane == kk
        idx_acc = jnp.where(ksel, ji, idx_acc)
        f_acc = tuple(jnp.where(ksel, fv, fa)
                      for fv, fa in zip((f1, f2, f3, f4, f5, f6), f_acc))
        return neg, idx_acc, f_acc

    init = (neg,
            jnp.zeros((TILE, KW), jnp.int32),
            tuple(jnp.zeros((TILE, KW), jnp.float32) for _ in range(6)))
    _, idx_acc, f_acc = jax.lax.fori_loop(0, KP1, body, init)

    idx_ref[...] = idx_acc
    for fi in range(6):
        feat_ref[:, fi * KW:(fi + 1) * KW] = f_acc[fi]


def _mlp_a_kernel(raw_ref, w1_ref, b1_ref, h1_ref, sums_ref):
    pid = pl.program_id(0)
    r = raw_ref[...]
    w = w1_ref[...]
    h = jax.lax.dot_general(r.astype(jnp.bfloat16), w.astype(jnp.bfloat16),
                            (((1,), (1,)), ((), ())),
                            preferred_element_type=jnp.float32) + b1_ref[...]
    h1_ref[...] = h
    rows = pid * ETILE + jax.lax.broadcasted_iota(jnp.int32, (ETILE, 1), 0)
    hm = jnp.where(rows < E, h, 0.0)
    st = jnp.concatenate([jnp.sum(hm, axis=0, keepdims=True),
                          jnp.sum(hm * hm, axis=0, keepdims=True)], axis=0)

    @pl.when(pid == 0)
    def _():
        sums_ref[...] = jnp.zeros_like(sums_ref)

    sums_ref[...] = sums_ref[...] + st


def _mlp_b_kernel(h1_ref, sums1_ref, g1_ref, be1_ref, w2_ref, b2_ref,
                  h2_ref, sums_ref):
    pid = pl.program_id(0)
    s = sums1_ref[...]
    m = s[0:1, :] / E
    v = s[1:2, :] / E - m * m
    h1 = h1_ref[...]
    a = jax.nn.relu(g1_ref[...] * (h1 - m) / jnp.sqrt(v + EPS_BN) + be1_ref[...])
    h2 = jax.lax.dot_general(a.astype(jnp.bfloat16),
                             w2_ref[...].astype(jnp.bfloat16),
                             (((1,), (1,)), ((), ())),
                             preferred_element_type=jnp.float32) + b2_ref[...]
    h2_ref[...] = h2
    rows = pid * ETILE + jax.lax.broadcasted_iota(jnp.int32, (ETILE, 1), 0)
    hm = jnp.where(rows < E, h2, 0.0)
    st = jnp.concatenate([jnp.sum(hm, axis=0, keepdims=True),
                          jnp.sum(hm * hm, axis=0, keepdims=True)], axis=0)

    @pl.when(pid == 0)
    def _():
        sums_ref[...] = jnp.zeros_like(sums_ref)

    sums_ref[...] = sums_ref[...] + st


def _mlp_c_kernel(h2_ref, sums2_ref, g2_ref, be2_ref, out_ref):
    s = sums2_ref[...]
    m = s[0:1, :] / E
    v = s[1:2, :] / E - m * m
    h2 = h2_ref[...]
    out_ref[...] = jax.nn.relu(
        g2_ref[...] * (h2 - m) / jnp.sqrt(v + EPS_BN) + be2_ref[...])


@functools.partial(jax.jit, static_argnames=())
def kernel(x, location_info, W1, b1, g1, be1, W2, b2, g2, be2, k):
    del k
    xft = x.T
    ltT = jnp.pad(location_info.T, ((0, 4), (0, 0)))

    idx_pad, feat_pad = pl.pallas_call(
        _knn_kernel,
        grid=(NT,),
        in_specs=[
            pl.BlockSpec((TILE, D_FEAT), lambda i: (i, 0)),
            pl.BlockSpec((D_FEAT, N), lambda i: (0, 0)),
            pl.BlockSpec((TILE, 4), lambda i: (i, 0)),
            pl.BlockSpec((8, N), lambda i: (0, 0)),
            pl.BlockSpec((N, 4), lambda i: (0, 0)),
        ],
        out_specs=[
            pl.BlockSpec((TILE, KW), lambda i: (i, 0)),
            pl.BlockSpec((TILE, 6 * KW), lambda i: (i, 0)),
        ],
        out_shape=[
            jax.ShapeDtypeStruct((NP, KW), jnp.int32),
            jax.ShapeDtypeStruct((NP, 6 * KW), jnp.float32),
        ],
    )(x, xft, location_info, ltT, location_info)

    src = idx_pad[:N, :KP1].reshape(-1)
    tgt = jnp.repeat(jnp.arange(N, dtype=src.dtype), KP1)
    edge_index = jnp.stack([src, tgt], axis=0)

    raw = (feat_pad[:N]
           .reshape(N, 6, KW)[:, :, :KP1]
           .transpose(0, 2, 1)
           .reshape(E, 6))

    h1, sums1 = pl.pallas_call(
        _mlp_a_kernel,
        grid=(EG,),
        in_specs=[
            pl.BlockSpec((ETILE, 6), lambda i: (i, 0)),
            pl.BlockSpec((18, 6), lambda i: (0, 0)),
            pl.BlockSpec((1, 18), lambda i: (0, 0)),
        ],
        out_specs=[
            pl.BlockSpec((ETILE, 18), lambda i: (i, 0)),
            pl.BlockSpec((2, 18), lambda i: (0, 0)),
        ],
        out_shape=[
            jax.ShapeDtypeStruct((E, 18), jnp.float32),
            jax.ShapeDtypeStruct((2, 18), jnp.float32),
        ],
    )(raw, W1, b1.reshape(1, 18))

    h2, sums2 = pl.pallas_call(
        _mlp_b_kernel,
        grid=(EG,),
        in_specs=[
            pl.BlockSpec((ETILE, 18), lambda i: (i, 0)),
            pl.BlockSpec((2, 18), lambda i: (0, 0)),
            pl.BlockSpec((1, 18), lambda i: (0, 0)),
            pl.BlockSpec((1, 18), lambda i: (0, 0)),
            pl.BlockSpec((16, 18), lambda i: (0, 0)),
            pl.BlockSpec((1, 16), lambda i: (0, 0)),
        ],
        out_specs=[
            pl.BlockSpec((ETILE, 16), lambda i: (i, 0)),
            pl.BlockSpec((2, 16), lambda i: (0, 0)),
        ],
        out_shape=[
            jax.ShapeDtypeStruct((E, 16), jnp.float32),
            jax.ShapeDtypeStruct((2, 16), jnp.float32),
        ],
    )(h1, sums1, g1.reshape(1, 18), be1.reshape(1, 18), W2, b2.reshape(1, 16))

    edge_attr = pl.pallas_call(
        _mlp_c_kernel,
        grid=(EG,),
        in_specs=[
            pl.BlockSpec((ETILE, 16), lambda i: (i, 0)),
            pl.BlockSpec((2, 16), lambda i: (0, 0)),
            pl.BlockSpec((1, 16), lambda i: (0, 0)),
            pl.BlockSpec((1, 16), lambda i: (0, 0)),
        ],
        out_specs=pl.BlockSpec((ETILE, 16), lambda i: (i, 0)),
        out_shape=jax.ShapeDtypeStruct((E, 16), jnp.float32),
    )(h2, sums2, g2.reshape(1, 16), be2.reshape(1, 16))

    return edge_index, edge_attr


# speedup vs baseline: 1.1201x; 1.1201x over previous
import functools

import jax
import jax.numpy as jnp
from jax.experimental import pallas as pl

N = 5000
D_FEAT = 256
KP1 = 33
E = N * KP1
TILE = 128
NT = (N + TILE - 1) // TILE
NP = NT * TILE
KW = 64
ETILE = 1024
EG = (E + ETILE - 1) // ETILE
EPS_BN = 1e-5
HIGHEST = jax.lax.Precision.HIGHEST


def _knn_kernel(xt_ref, xft_ref, lt_ref, ltT_ref, lf_ref, idx_ref, feat_ref):
    xt = xt_ref[...]
    xft = xft_ref[...]
    lt = lt_ref[...]
    ltT = ltT_ref[...]
    lf = lf_ref[...]

    px_r = ltT[0:1, :]
    py_r = ltT[1:2, :]

    sq_r = px_r * px_r + py_r * py_r
    pxt = lt[:, 0:1]
    pyt = lt[:, 1:2]
    sq_t = pxt * pxt + pyt * pyt

    pxt_b = pxt.astype(jnp.bfloat16).astype(jnp.float32)
    pyt_b = pyt.astype(jnp.bfloat16).astype(jnp.float32)
    pxr_b = px_r.astype(jnp.bfloat16).astype(jnp.float32)
    pyr_b = py_r.astype(jnp.bfloat16).astype(jnp.float32)
    g = pxt_b * pxr_b + pyt_b * pyr_b
    d = sq_t + sq_r - 2.0 * g
    neg = -d

    S = jax.lax.dot_general(xt, xft, (((1,), (0,)), ((), ())),
                            precision=HIGHEST)

    na = jnp.maximum(
        jnp.sqrt(jnp.sum(xt * xt, axis=1, keepdims=True)), 1e-8)
    xft_sq = xft * xft
    ns_col = jax.lax.dot_general(xft_sq, jnp.ones((D_FEAT, 1), jnp.float32),
                                 (((0,), (0,)), ((), ())))
    nb_col = jnp.maximum(jnp.sqrt(ns_col), 1e-8)

    tab5 = jnp.concatenate([lf, nb_col], axis=1)
    tab_hi = tab5.astype(jnp.bfloat16)
    r1_ = tab5 - tab_hi.astype(jnp.float32)
    tab_mid = r1_.astype(jnp.bfloat16)
    tab_lo = (r1_ - tab_mid.astype(jnp.float32)).astype(jnp.bfloat16)
    tab = jnp.concatenate([tab_hi, tab_mid, tab_lo], axis=1)

    cols = jax.lax.broadcasted_iota(jnp.int32, (TILE, N), 1)
    lane = jax.lax.broadcasted_iota(jnp.int32, (TILE, KW), 1)

    tx, ty = pxt, pyt
    tw = lt[:, 2:3]
    th = lt[:, 3:4]

    BIG = jnp.int32(1 << 30)
    NEG = jnp.float32(-3.0e38)

    def body(kk, carry):
        neg, idx_acc, f_acc = carry
        m = jnp.max(neg, axis=1, keepdims=True)
        tiebreak = jnp.where(neg == m, cols, BIG)
        ji = jnp.min(tiebreak, axis=1, keepdims=True)
        mask = cols == ji
        neg = jnp.where(mask, NEG, neg)

        oh = mask.astype(jnp.bfloat16)
        res = jax.lax.dot_general(oh, tab, (((1,), (0,)), ((), ())),
                                  preferred_element_type=jnp.float32)

        def _rec(c):
            return (res[:, c:c + 1] + res[:, c + 5:c + 6]) + res[:, c + 10:c + 11]

        sx = _rec(0)
        sy = _rec(1)
        sw = _rec(2)
        sh = _rec(3)
        nb = _rec(4)
        dot = jnp.sum(jnp.where(mask, S, 0.0), axis=1, keepdims=True)

        hsum = sh + th + 1e-8
        f1 = 2.0 * (sx - tx) / hsum
        f2 = 2.0 * (sy - ty) / hsum
        f3 = jnp.log(sh / (th + 1e-8))
        f4 = jnp.log(sw / (tw + 1e-8))

        sx1 = sx - sw / 2
        sy1 = sy - sh / 2
        sx2 = sx + sw / 2
        sy2 = sy + sh / 2
        tx1 = tx - tw / 2
        ty1 = ty - th / 2
        tx2 = tx + tw / 2
        ty2 = ty + th / 2
        ix1 = jnp.maximum(sx1, tx1)
        iy1 = jnp.maximum(sy1, ty1)
        ix2 = jnp.minimum(sx2, tx2)
        iy2 = jnp.minimum(sy2, ty2)
        inter = jnp.clip(ix2 - ix1, 0.0, None) * jnp.clip(iy2 - iy1, 0.0, None)
        sa = sw * sh
        ta = tw * th
        union = sa + ta - inter
        ixc = (ix1 + ix2) / 2
        iyc = (iy1 + iy2) / 2
        uxc = (sx * sa + tx * ta) / (union + 1e-8)
        uyc = (sy * sa + ty * ta) / (union + 1e-8)
        dist = jnp.sqrt((ixc - uxc) ** 2 + (iyc - uyc) ** 2)
        f5 = inter / (union + 1e-8) - dist ** 2 / (union + 1e-8)

        f6 = dot / (na * nb)

        ksel = lane == kk
        idx_acc = jnp.where(ksel, ji, idx_acc)
        f_acc = tuple(jnp.where(ksel, fv, fa)
                      for fv, fa in zip((f1, f2, f3, f4, f5, f6), f_acc))
        return neg, idx_acc, f_acc

    init = (neg,
            jnp.zeros((TILE, KW), jnp.int32),
            tuple(jnp.zeros((TILE, KW), jnp.float32) for _ in range(6)))
    _, idx_acc, f_acc = jax.lax.fori_loop(0, KP1, body, init)

    idx_ref[...] = idx_acc
    for fi in range(6):
        feat_ref[:, fi * KW:(fi + 1) * KW] = f_acc[fi]


def _mlp_a_kernel(raw_ref, w1_ref, b1_ref, h1_ref, sums_ref):
    pid = pl.program_id(0)
    r = raw_ref[...]
    w = w1_ref[...]
    h = jax.lax.dot_general(r.astype(jnp.bfloat16), w.astype(jnp.bfloat16),
                            (((1,), (1,)), ((), ())),
                            preferred_element_type=jnp.float32) + b1_ref[...]
    h1_ref[...] = h
    rows = pid * ETILE + jax.lax.broadcasted_iota(jnp.int32, (ETILE, 1), 0)
    hm = jnp.where(rows < E, h, 0.0)
    st = jnp.concatenate([jnp.sum(hm, axis=0, keepdims=True),
                          jnp.sum(hm * hm, axis=0, keepdims=True)], axis=0)

    @pl.when(pid == 0)
    def _():
        sums_ref[...] = jnp.zeros_like(sums_ref)

    sums_ref[...] = sums_ref[...] + st


def _mlp_b_kernel(h1_ref, sums1_ref, g1_ref, be1_ref, w2_ref, b2_ref,
                  h2_ref, sums_ref):
    pid = pl.program_id(0)
    s = sums1_ref[...]
    m = s[0:1, :] / E
    v = s[1:2, :] / E - m * m
    h1 = h1_ref[...]
    a = jax.nn.relu(g1_ref[...] * (h1 - m) / jnp.sqrt(v + EPS_BN) + be1_ref[...])
    h2 = jax.lax.dot_general(a.astype(jnp.bfloat16),
                             w2_ref[...].astype(jnp.bfloat16),
                             (((1,), (1,)), ((), ())),
                             preferred_element_type=jnp.float32) + b2_ref[...]
    h2_ref[...] = h2
    rows = pid * ETILE + jax.lax.broadcasted_iota(jnp.int32, (ETILE, 1), 0)
    hm = jnp.where(rows < E, h2, 0.0)
    st = jnp.concatenate([jnp.sum(hm, axis=0, keepdims=True),
                          jnp.sum(hm * hm, axis=0, keepdims=True)], axis=0)

    @pl.when(pid == 0)
    def _():
        sums_ref[...] = jnp.zeros_like(sums_ref)

    sums_ref[...] = sums_ref[...] + st


def _mlp_c_kernel(h2_ref, sums2_ref, g2_ref, be2_ref, out_ref):
    s = sums2_ref[...]
    m = s[0:1, :] / E
    v = s[1:2, :] / E - m * m
    h2 = h2_ref[...]
    out_ref[...] = jax.nn.relu(
        g2_ref[...] * (h2 - m) / jnp.sqrt(v + EPS_BN) + be2_ref[...])


@functools.partial(jax.jit, static_argnames=())
def kernel(x, location_info, W1, b1, g1, be1, W2, b2, g2, be2, k):
    del k
    xft = x.T
    ltT = jnp.pad(location_info.T, ((0, 4), (0, 0)))

    idx_pad, feat_pad = pl.pallas_call(
        _knn_kernel,
        grid=(NT,),
        in_specs=[
            pl.BlockSpec((TILE, D_FEAT), lambda i: (i, 0)),
            pl.BlockSpec((D_FEAT, N), lambda i: (0, 0)),
            pl.BlockSpec((TILE, 4), lambda i: (i, 0)),
            pl.BlockSpec((8, N), lambda i: (0, 0)),
            pl.BlockSpec((N, 4), lambda i: (0, 0)),
        ],
        out_specs=[
            pl.BlockSpec((TILE, KW), lambda i: (i, 0)),
            pl.BlockSpec((TILE, 6 * KW), lambda i: (i, 0)),
        ],
        out_shape=[
            jax.ShapeDtypeStruct((NP, KW), jnp.int32),
            jax.ShapeDtypeStruct((NP, 6 * KW), jnp.float32),
        ],
    )(x, xft, location_info, ltT, location_info)

    src = idx_pad[:N, :KP1].reshape(-1)
    tgt = jnp.repeat(jnp.arange(N, dtype=src.dtype), KP1)
    edge_index = jnp.stack([src, tgt], axis=0)

    raw = (feat_pad[:N]
           .reshape(N, 6, KW)[:, :, :KP1]
           .transpose(0, 2, 1)
           .reshape(E, 6))

    h1, sums1 = pl.pallas_call(
        _mlp_a_kernel,
        grid=(EG,),
        in_specs=[
            pl.BlockSpec((ETILE, 6), lambda i: (i, 0)),
            pl.BlockSpec((18, 6), lambda i: (0, 0)),
            pl.BlockSpec((1, 18), lambda i: (0, 0)),
        ],
        out_specs=[
            pl.BlockSpec((ETILE, 18), lambda i: (i, 0)),
            pl.BlockSpec((2, 18), lambda i: (0, 0)),
        ],
        out_shape=[
            jax.ShapeDtypeStruct((E, 18), jnp.float32),
            jax.ShapeDtypeStruct((2, 18), jnp.float32),
        ],
    )(raw, W1, b1.reshape(1, 18))

    h2, sums2 = pl.pallas_call(
        _mlp_b_kernel,
        grid=(EG,),
        in_specs=[
            pl.BlockSpec((ETILE, 18), lambda i: (i, 0)),
            pl.BlockSpec((2, 18), lambda i: (0, 0)),
            pl.BlockSpec((1, 18), lambda i: (0, 0)),
            pl.BlockSpec((1, 18), lambda i: (0, 0)),
            pl.BlockSpec((16, 18), lambda i: (0, 0)),
            pl.BlockSpec((1, 16), lambda i: (0, 0)),
        ],
        out_specs=[
            pl.BlockSpec((ETILE, 16), lambda i: (i, 0)),
            pl.BlockSpec((2, 16), lambda i: (0, 0)),
        ],
        out_shape=[
            jax.ShapeDtypeStruct((E, 16), jnp.float32),
            jax.ShapeDtypeStruct((2, 16), jnp.float32),
        ],
    )(h1, sums1, g1.reshape(1, 18), be1.reshape(1, 18), W2, b2.reshape(1, 16))

    edge_attr = pl.pallas_call(
        _mlp_c_kernel,
        grid=(EG,),
        in_specs=[
            pl.BlockSpec((ETILE, 16), lambda i: (i, 0)),
            pl.BlockSpec((2, 16), lambda i: (0, 0)),
            pl.BlockSpec((1, 16), lambda i: (0, 0)),
            pl.BlockSpec((1, 16), lambda i: (0, 0)),
        ],
        out_specs=pl.BlockSpec((ETILE, 16), lambda i: (i, 0)),
        out_shape=jax.ShapeDtypeStruct((E, 16), jnp.float32),
    )(h2, sums2, g2.reshape(1, 16), be2.reshape(1, 16))

    return edge_index, edge_attr


# speedup vs baseline: 1.2892x; 1.1510x over previous
import functools

import jax
import jax.numpy as jnp
from jax.experimental import pallas as pl

N = 5000
D_FEAT = 256
KP1 = 33
E = N * KP1
TILE = 256
NT = (N + TILE - 1) // TILE
NP = NT * TILE
KW = 64
ETILE = 2048
EG = (E + ETILE - 1) // ETILE
EPS_BN = 1e-5
HIGHEST = jax.lax.Precision.HIGHEST


def _knn_kernel(xt_ref, xft_ref, lt_ref, ltT_ref, lf_ref, idx_ref, feat_ref):
    xt = xt_ref[...]
    xft = xft_ref[...]
    lt = lt_ref[...]
    ltT = ltT_ref[...]
    lf = lf_ref[...]

    px_r = ltT[0:1, :]
    py_r = ltT[1:2, :]

    sq_r = px_r * px_r + py_r * py_r
    pxt = lt[:, 0:1]
    pyt = lt[:, 1:2]
    sq_t = pxt * pxt + pyt * pyt

    pxt_b = pxt.astype(jnp.bfloat16).astype(jnp.float32)
    pyt_b = pyt.astype(jnp.bfloat16).astype(jnp.float32)
    pxr_b = px_r.astype(jnp.bfloat16).astype(jnp.float32)
    pyr_b = py_r.astype(jnp.bfloat16).astype(jnp.float32)
    g = pxt_b * pxr_b + pyt_b * pyr_b
    d = sq_t + sq_r - 2.0 * g
    neg = -d

    S = jax.lax.dot_general(xt, xft, (((1,), (0,)), ((), ())),
                            precision=HIGHEST)

    na = jnp.maximum(
        jnp.sqrt(jnp.sum(xt * xt, axis=1, keepdims=True)), 1e-8)
    xft_sq = xft * xft
    ns_col = jax.lax.dot_general(xft_sq, jnp.ones((D_FEAT, 1), jnp.float32),
                                 (((0,), (0,)), ((), ())))
    nb_col = jnp.maximum(jnp.sqrt(ns_col), 1e-8)

    tab5 = jnp.concatenate([lf, nb_col], axis=1)
    tab_hi = tab5.astype(jnp.bfloat16)
    r1_ = tab5 - tab_hi.astype(jnp.float32)
    tab_mid = r1_.astype(jnp.bfloat16)
    tab_lo = (r1_ - tab_mid.astype(jnp.float32)).astype(jnp.bfloat16)
    tab = jnp.concatenate([tab_hi, tab_mid, tab_lo], axis=1)

    cols = jax.lax.broadcasted_iota(jnp.int32, (TILE, N), 1)
    lane = jax.lax.broadcasted_iota(jnp.int32, (TILE, KW), 1)

    tx, ty = pxt, pyt
    tw = lt[:, 2:3]
    th = lt[:, 3:4]

    BIG = jnp.int32(1 << 30)
    NEG = jnp.float32(-3.0e38)

    def body(kk, carry):
        neg, idx_acc, f_acc = carry
        m = jnp.max(neg, axis=1, keepdims=True)
        tiebreak = jnp.where(neg == m, cols, BIG)
        ji = jnp.min(tiebreak, axis=1, keepdims=True)
        mask = cols == ji
        neg = jnp.where(mask, NEG, neg)

        oh = mask.astype(jnp.bfloat16)
        res = jax.lax.dot_general(oh, tab, (((1,), (0,)), ((), ())),
                                  preferred_element_type=jnp.float32)

        def _rec(c):
            return (res[:, c:c + 1] + res[:, c + 5:c + 6]) + res[:, c + 10:c + 11]

        sx = _rec(0)
        sy = _rec(1)
        sw = _rec(2)
        sh = _rec(3)
        nb = _rec(4)
        dot = jnp.sum(jnp.where(mask, S, 0.0), axis=1, keepdims=True)

        hsum = sh + th + 1e-8
        f1 = 2.0 * (sx - tx) / hsum
        f2 = 2.0 * (sy - ty) / hsum
        f3 = jnp.log(sh / (th + 1e-8))
        f4 = jnp.log(sw / (tw + 1e-8))

        sx1 = sx - sw / 2
        sy1 = sy - sh / 2
        sx2 = sx + sw / 2
        sy2 = sy + sh / 2
        tx1 = tx - tw / 2
        ty1 = ty - th / 2
        tx2 = tx + tw / 2
        ty2 = ty + th / 2
        ix1 = jnp.maximum(sx1, tx1)
        iy1 = jnp.maximum(sy1, ty1)
        ix2 = jnp.minimum(sx2, tx2)
        iy2 = jnp.minimum(sy2, ty2)
        inter = jnp.clip(ix2 - ix1, 0.0, None) * jnp.clip(iy2 - iy1, 0.0, None)
        sa = sw * sh
        ta = tw * th
        union = sa + ta - inter
        ixc = (ix1 + ix2) / 2
        iyc = (iy1 + iy2) / 2
        uxc = (sx * sa + tx * ta) / (union + 1e-8)
        uyc = (sy * sa + ty * ta) / (union + 1e-8)
        dist = jnp.sqrt((ixc - uxc) ** 2 + (iyc - uyc) ** 2)
        f5 = inter / (union + 1e-8) - dist ** 2 / (union + 1e-8)

        f6 = dot / (na * nb)

        ksel = lane == kk
        idx_acc = jnp.where(ksel, ji, idx_acc)
        f_acc = tuple(jnp.where(ksel, fv, fa)
                      for fv, fa in zip((f1, f2, f3, f4, f5, f6), f_acc))
        return neg, idx_acc, f_acc

    init = (neg,
            jnp.zeros((TILE, KW), jnp.int32),
            tuple(jnp.zeros((TILE, KW), jnp.float32) for _ in range(6)))
    _, idx_acc, f_acc = jax.lax.fori_loop(0, KP1, body, init)

    idx_ref[...] = idx_acc
    for fi in range(6):
        feat_ref[:, fi * KW:(fi + 1) * KW] = f_acc[fi]


def _mlp_a_kernel(raw_ref, w1_ref, b1_ref, h1_ref, sums_ref):
    pid = pl.program_id(0)
    r = raw_ref[...]
    w = w1_ref[...]
    h = jax.lax.dot_general(r.astype(jnp.bfloat16), w.astype(jnp.bfloat16),
                            (((1,), (1,)), ((), ())),
                            preferred_element_type=jnp.float32) + b1_ref[...]
    h1_ref[...] = h
    rows = pid * ETILE + jax.lax.broadcasted_iota(jnp.int32, (ETILE, 1), 0)
    hm = jnp.where(rows < E, h, 0.0)
    st = jnp.concatenate([jnp.sum(hm, axis=0, keepdims=True),
                          jnp.sum(hm * hm, axis=0, keepdims=True)], axis=0)

    @pl.when(pid == 0)
    def _():
        sums_ref[...] = jnp.zeros_like(sums_ref)

    sums_ref[...] = sums_ref[...] + st


def _mlp_b_kernel(h1_ref, sums1_ref, g1_ref, be1_ref, w2_ref, b2_ref,
                  h2_ref, sums_ref):
    pid = pl.program_id(0)
    s = sums1_ref[...]
    m = s[0:1, :] / E
    v = s[1:2, :] / E - m * m
    h1 = h1_ref[...]
    a = jax.nn.relu(g1_ref[...] * (h1 - m) / jnp.sqrt(v + EPS_BN) + be1_ref[...])
    h2 = jax.lax.dot_general(a.astype(jnp.bfloat16),
                             w2_ref[...].astype(jnp.bfloat16),
                             (((1,), (1,)), ((), ())),
                             preferred_element_type=jnp.float32) + b2_ref[...]
    h2_ref[...] = h2
    rows = pid * ETILE + jax.lax.broadcasted_iota(jnp.int32, (ETILE, 1), 0)
    hm = jnp.where(rows < E, h2, 0.0)
    st = jnp.concatenate([jnp.sum(hm, axis=0, keepdims=True),
                          jnp.sum(hm * hm, axis=0, keepdims=True)], axis=0)

    @pl.when(pid == 0)
    def _():
        sums_ref[...] = jnp.zeros_like(sums_ref)

    sums_ref[...] = sums_ref[...] + st


def _mlp_c_kernel(h2_ref, sums2_ref, g2_ref, be2_ref, out_ref):
    s = sums2_ref[...]
    m = s[0:1, :] / E
    v = s[1:2, :] / E - m * m
    h2 = h2_ref[...]
    out_ref[...] = jax.nn.relu(
        g2_ref[...] * (h2 - m) / jnp.sqrt(v + EPS_BN) + be2_ref[...])


@functools.partial(jax.jit, static_argnames=())
def kernel(x, location_info, W1, b1, g1, be1, W2, b2, g2, be2, k):
    del k
    xft = x.T
    ltT = jnp.pad(location_info.T, ((0, 4), (0, 0)))

    idx_pad, feat_pad = pl.pallas_call(
        _knn_kernel,
        grid=(NT,),
        in_specs=[
            pl.BlockSpec((TILE, D_FEAT), lambda i: (i, 0)),
            pl.BlockSpec((D_FEAT, N), lambda i: (0, 0)),
            pl.BlockSpec((TILE, 4), lambda i: (i, 0)),
            pl.BlockSpec((8, N), lambda i: (0, 0)),
            pl.BlockSpec((N, 4), lambda i: (0, 0)),
        ],
        out_specs=[
            pl.BlockSpec((TILE, KW), lambda i: (i, 0)),
            pl.BlockSpec((TILE, 6 * KW), lambda i: (i, 0)),
        ],
        out_shape=[
            jax.ShapeDtypeStruct((NP, KW), jnp.int32),
            jax.ShapeDtypeStruct((NP, 6 * KW), jnp.float32),
        ],
    )(x, xft, location_info, ltT, location_info)

    src = idx_pad[:N, :KP1].reshape(-1)
    tgt = jnp.repeat(jnp.arange(N, dtype=src.dtype), KP1)
    edge_index = jnp.stack([src, tgt], axis=0)

    raw = (feat_pad[:N]
           .reshape(N, 6, KW)[:, :, :KP1]
           .transpose(0, 2, 1)
           .reshape(E, 6))

    h1, sums1 = pl.pallas_call(
        _mlp_a_kernel,
        grid=(EG,),
        in_specs=[
            pl.BlockSpec((ETILE, 6), lambda i: (i, 0)),
            pl.BlockSpec((18, 6), lambda i: (0, 0)),
            pl.BlockSpec((1, 18), lambda i: (0, 0)),
        ],
        out_specs=[
            pl.BlockSpec((ETILE, 18), lambda i: (i, 0)),
            pl.BlockSpec((2, 18), lambda i: (0, 0)),
        ],
        out_shape=[
            jax.ShapeDtypeStruct((E, 18), jnp.float32),
            jax.ShapeDtypeStruct((2, 18), jnp.float32),
        ],
    )(raw, W1, b1.reshape(1, 18))

    h2, sums2 = pl.pallas_call(
        _mlp_b_kernel,
        grid=(EG,),
        in_specs=[
            pl.BlockSpec((ETILE, 18), lambda i: (i, 0)),
            pl.BlockSpec((2, 18), lambda i: (0, 0)),
            pl.BlockSpec((1, 18), lambda i: (0, 0)),
            pl.BlockSpec((1, 18), lambda i: (0, 0)),
            pl.BlockSpec((16, 18), lambda i: (0, 0)),
            pl.BlockSpec((1, 16), lambda i: (0, 0)),
        ],
        out_specs=[
            pl.BlockSpec((ETILE, 16), lambda i: (i, 0)),
            pl.BlockSpec((2, 16), lambda i: (0, 0)),
        ],
        out_shape=[
            jax.ShapeDtypeStruct((E, 16), jnp.float32),
            jax.ShapeDtypeStruct((2, 16), jnp.float32),
        ],
    )(h1, sums1, g1.reshape(1, 18), be1.reshape(1, 18), W2, b2.reshape(1, 16))

    edge_attr = pl.pallas_call(
        _mlp_c_kernel,
        grid=(EG,),
        in_specs=[
            pl.BlockSpec((ETILE, 16), lambda i: (i, 0)),
            pl.BlockSpec((2, 16), lambda i: (0, 0)),
            pl.BlockSpec((1, 16), lambda i: (0, 0)),
            pl.BlockSpec((1, 16), lambda i: (0, 0)),
        ],
        out_specs=pl.BlockSpec((ETILE, 16), lambda i: (i, 0)),
        out_shape=jax.ShapeDtypeStruct((E, 16), jnp.float32),
    )(h2, sums2, g2.reshape(1, 16), be2.reshape(1, 16))

    return edge_index, edge_attr
